# baseline (speedup 1.0000x reference)
import functools

import jax
import jax.numpy as jnp
from jax import lax
from jax.experimental import pallas as pl
from jax.experimental.pallas import tpu as pltpu
from jax.experimental.pallas import tpu_sc as plsc

EPS = 1e-10
K_VAL = 250
WEIGHT_VAL = 8.0

V = 100000
NV = V // 16
CAP = 3072
MININT = -2147483648

TV = 2048

_STAGE = 4


def _bias_body(e_ref, w_ref, o_ref):
    e = e_ref[...]
    w = w_ref[...]
    t1 = jnp.sum(w * w, axis=1)
    t3 = jnp.sum(e * e, axis=1)
    t2 = lax.dot_general(e, w, (((1,), (1,)), ((), ())),
                         preferred_element_type=jnp.float32)
    o_ref[...] = (2.0 * WEIGHT_VAL) * t2 \
        - WEIGHT_VAL * t1[None, :] - WEIGHT_VAL * t3[:, None]


def _bias_pallas(cur_embeds, embed_weight):
    R, E = cur_embeds.shape
    Vn = embed_weight.shape[0]
    return pl.pallas_call(
        _bias_body,
        grid=(pl.cdiv(Vn, TV),),
        in_specs=[
            pl.BlockSpec((R, E), lambda i: (0, 0)),
            pl.BlockSpec((TV, E), lambda i: (i, 0)),
        ],
        out_specs=pl.BlockSpec((R, TV), lambda i: (0, i)),
        out_shape=jax.ShapeDtypeStruct((R, Vn), jnp.float32),
    )(cur_embeds, embed_weight)


def _sc_sampler(logits2d, gxflat, hrow_arr, cur_arr, gmb_pad, embed_weight):
    mesh = plsc.VectorSubcoreMesh(core_axis_name="c", subcore_axis_name="s")

    @functools.partial(
        pl.kernel,
        mesh=mesh,
        compiler_params=pltpu.CompilerParams(needs_layout_passes=False),
        out_type=jax.ShapeDtypeStruct((256, 64), jnp.float32),
        scratch_types=[
            pltpu.VMEM((V,), jnp.float32),
            pltpu.VMEM((8192,), jnp.int32),
            pltpu.VMEM((CAP + 16,), jnp.int32),
            pltpu.VMEM((CAP + 16,), jnp.int32),
            pltpu.VMEM((272,), jnp.int32),
            pltpu.VMEM((272,), jnp.int32),
            pltpu.VMEM((128,), jnp.int32),
            pltpu.VMEM((128,), jnp.int32),
            pltpu.VMEM((128,), jnp.float32),
            pltpu.VMEM((128,), jnp.float32),
            pltpu.VMEM((256,), jnp.float32),
            pltpu.VMEM((272,), jnp.int32),
            pltpu.VMEM((272,), jnp.int32),
            pltpu.VMEM((32,), jnp.int32),
            pltpu.VMEM((8, 64), jnp.float32),
            pltpu.SMEM((8,), jnp.int32),
            pltpu.SemaphoreType.DMA,
        ],
    )
    def sck(lg_hbm, gx_hbm, hr_hbm, cu_hbm, gm_hbm, em_hbm, out_hbm,
            row_buf, hist, cks, cidx, Kb, Ib, ixa, ixb, gxa, gxb,
            gmb, hrv, crv, tokv, embr, cnt, sem):
        wid = lax.axis_index("s") * 2 + lax.axis_index("c")
        lane = lax.broadcasted_iota(jnp.int32, (16,), 0)
        zeros16 = jnp.zeros((16,), jnp.int32)
        ones16 = jnp.ones((16,), jnp.int32)
        pltpu.sync_copy(hr_hbm, hrv.at[pl.ds(0, 256)])
        pltpu.sync_copy(cu_hbm, crv.at[pl.ds(0, 256)])

        def keyize(v):
            b = lax.bitcast_convert_type(v, jnp.int32)
            m = lax.shift_right_logical(lax.shift_right_arithmetic(b, 31), 1)
            ks = b ^ m
            t9 = lax.shift_right_logical(ks, 23) ^ 256
            return ks, t9

        def row_fn(j, carry):
            toks, guess = carry
            row = wid * 8 + j
            hrow = hrv[pl.ds(row, 16)][0]
            pltpu.sync_copy(lg_hbm.at[hrow], row_buf)
            pltpu.sync_copy(gm_hbm.at[row], gmb)

            def zb(i, _):
                for u in range(8):
                    hist[pl.ds((i * 8 + u) * 16, 16)] = zeros16
                return 0
            lax.fori_loop(0, 64, zb, 0, unroll=False)

            bgv = jnp.full((16,), guess, jnp.int32)

            def fused_g(i, offv):
                kss, msks = [], []
                for u in range(4):
                    v = row_buf[pl.ds((i * 4 + u) * 16, 16)]
                    ks, t9 = keyize(v)
                    plsc.addupdate_scatter(hist, [t9 * 16 + lane], ones16)
                    kss.append(ks)
                    msks.append(t9 >= bgv)
                anym = (msks[0] | msks[1]) | (msks[2] | msks[3])

                def dostore(_):
                    o = offv[0]
                    for u in range(4):
                        oc = jnp.minimum(o, CAP)
                        plsc.store_compressed(cks.at[pl.ds(oc, 16)], kss[u],
                                              mask=msks[u])
                        plsc.store_compressed(cidx.at[pl.ds(oc, 16)],
                                              (i * 4 + u) * 16 + lane,
                                              mask=msks[u])
                        o = o + jnp.sum(msks[u].astype(jnp.int32))
                    return 0
                lax.cond(jnp.any(anym), dostore, lambda _: 0, 0)
                upd = (plsc.all_reduce_population_count(msks[0])
                       + plsc.all_reduce_population_count(msks[1])
                       + plsc.all_reduce_population_count(msks[2])
                       + plsc.all_reduce_population_count(msks[3]))
                return offv + upd
            offv = lax.fori_loop(0, NV // 4, fused_g, zeros16)
            for i4 in (NV // 4 * 4, NV // 4 * 4 + 1):
                v = row_buf[pl.ds(i4 * 16, 16)]
                ks, t9 = keyize(v)
                plsc.addupdate_scatter(hist, [t9 * 16 + lane], ones16)
                msk = t9 >= bgv
                oc = jnp.minimum(offv[0], CAP)
                plsc.store_compressed(cks.at[pl.ds(oc, 16)], ks, mask=msk)
                plsc.store_compressed(cidx.at[pl.ds(oc, 16)], i4 * 16 + lane,
                                      mask=msk)
                offv = offv + plsc.all_reduce_population_count(msk)

            def ga(t, c):
                cum, gst, gcb = c
                g = 31 - t
                acc = zeros16
                for bb in range(16):
                    acc = acc + hist[pl.ds((g * 16 + bb) * 16, 16)]
                tg = jnp.sum(acc)
                cumn = cum + tg
                hit = (cum < K_VAL) & (cumn >= K_VAL)
                gst = jnp.where(hit, g, gst)
                gcb = jnp.where(hit, cum, gcb)
                return cumn, gst, gcb
            _, gst, gcb = lax.fori_loop(0, 32, ga, (0, 0, 0))

            def gb(t, c):
                cum, bst, cab = c
                bb = gst * 16 + (15 - t)
                cc = jnp.sum(hist[pl.ds(bb * 16, 16)])
                cumn = cum + cc
                hit = (cum < K_VAL) & (cumn >= K_VAL)
                bst = jnp.where(hit, bb, bst)
                cab = jnp.where(hit, cum, cab)
                return cumn, bst, cab
            _, bstar, cnt_above = lax.fori_loop(0, 16, gb, (gcb, 0, 0))

            bsv = jnp.full((16,), bstar, jnp.int32)

            def cp_exact(i, offv2):
                v = row_buf[pl.ds(i * 16, 16)]
                ks, t9 = keyize(v)
                msk = t9 >= bsv
                offc = jnp.minimum(offv2[0], CAP)
                plsc.store_compressed(cks.at[pl.ds(offc, 16)], ks, mask=msk)
                plsc.store_compressed(cidx.at[pl.ds(offc, 16)], i * 16 + lane,
                                      mask=msk)
                return offv2 + plsc.all_reduce_population_count(msk)

            bad = (bstar < guess) | (offv[0] > CAP)
            cnt_f = lax.cond(
                bad,
                lambda _: lax.fori_loop(0, NV, cp_exact, zeros16, unroll=4)[0],
                lambda _: offv[0], 0)
            C = jnp.minimum(cnt_f, CAP)
            Cv = jnp.full((16,), C, jnp.int32)
            nvc = lax.shift_right_logical(C + 15, 4)

            pfx = bstar
            pshift = 23
            cab = cnt_above
            for width in (5, 5, 5, 4, 4):
                shift = pshift - width
                nb = 1 << width
                for u in range(nb):
                    hist[pl.ds(u * 16, 16)] = zeros16
                pfxv = jnp.full((16,), pfx, jnp.int32)

                def rf(i, _, pfxv=pfxv, pshift=pshift, shift=shift, nb=nb):
                    ks = cks[pl.ds(i * 16, 16)]
                    ku = ks ^ MININT
                    gi = (i * 16 + lane) < Cv
                    match = (lax.shift_right_logical(ku, pshift) == pfxv) & gi
                    bins = lax.shift_right_logical(ku, shift) & (nb - 1)
                    plsc.addupdate_scatter(hist, [bins * 16 + lane], ones16,
                                           mask=match)
                    return 0
                lax.fori_loop(0, nvc, rf, 0)

                def sc2(t, c, nb=nb):
                    cum, bst, cab2 = c
                    bb = (nb - 1) - t
                    cc = jnp.sum(hist[pl.ds(bb * 16, 16)])
                    cumn = cum + cc
                    hit = (cum < K_VAL) & (cumn >= K_VAL)
                    bst = jnp.where(hit, bb, bst)
                    cab2 = jnp.where(hit, cum, cab2)
                    return cumn, bst, cab2
                _, bst, cab = lax.fori_loop(0, nb, sc2, (cab, 0, 0))
                pfx = lax.shift_left(pfx, width) | bst
                pshift = shift
            kth_ks = pfx ^ MININT
            need = K_VAL - cab

            for t in range(17):
                Kb[pl.ds(t * 16, 16)] = jnp.full((16,), MININT, jnp.int32)
                Ib[pl.ds(t * 16, 16)] = zeros16
            cnt[1] = 0
            cnt[2] = 0
            kthv = jnp.full((16,), kth_ks, jnp.int32)

            def ex(i, _):
                ks = cks[pl.ds(i * 16, 16)]
                iv = cidx[pl.ds(i * 16, 16)]
                gi = (i * 16 + lane) < Cv
                gt = (ks > kthv) & gi
                eq = (ks == kthv) & gi
                eqi = eq.astype(jnp.int32)
                pre = plsc.cumsum(eqi) - eqi
                take = eq & ((pre + cnt[2]) < need)
                fm = gt | take
                off = cnt[1]
                plsc.store_compressed(Kb.at[pl.ds(off, 16)], ks, mask=fm)
                plsc.store_compressed(Ib.at[pl.ds(off, 16)], iv, mask=fm)
                cnt[1] = off + jnp.sum(fm.astype(jnp.int32))
                cnt[2] = cnt[2] + jnp.sum(eqi)
                return 0
            lax.fori_loop(0, nvc, ex, 0, unroll=False)

            gxbase = jnp.full((16,), hrow * V, jnp.int32)
            for t in range(8):
                ixa[pl.ds(t * 16, 16)] = Ib[pl.ds(t * 16, 16)] + gxbase
            for t in range(8, 16):
                ixb[pl.ds((t - 8) * 16, 16)] = Ib[pl.ds(t * 16, 16)] + gxbase
            cpa = pltpu.async_copy(gx_hbm.at[ixa], gxa, sem)
            cpb = pltpu.async_copy(gx_hbm.at[ixb], gxb, sem)

            curv = jnp.full((16,), crv[pl.ds(row, 16)][0], jnp.int32)
            best = jnp.full((16,), -jnp.inf, jnp.float32)
            bsr = jnp.full((16,), 1 << 30, jnp.int32)
            btk = zeros16
            big = jnp.int32(1 << 30)
            for qb in range(4):
                KQ = [Kb[pl.ds((qb * 4 + q) * 16, 16)] for q in range(4)]
                IQ = [Ib[pl.ds((qb * 4 + q) * 16, 16)] for q in range(4)]

                def apb(t, rn, KQ=KQ, IQ=IQ):
                    kpv = jnp.full((16,), Kb[pl.ds(t, 16)][0], jnp.int32)
                    ipv = jnp.full((16,), Ib[pl.ds(t, 16)][0], jnp.int32)
                    out = []
                    for q in range(4):
                        gt = (kpv > KQ[q]).astype(jnp.int32)
                        eq = ((kpv == KQ[q]) & (ipv < IQ[q])).astype(jnp.int32)
                        out.append(rn[q] + gt + eq)
                    return tuple(out)
                rn = lax.fori_loop(0, 256, apb, (zeros16,) * 4)
                if qb == 0:
                    cpa.wait()
                    cpb.wait()
                for q in range(4):
                    t = qb * 4 + q
                    if t < 8:
                        gxt = gxa[pl.ds(t * 16, 16)]
                    else:
                        gxt = gxb[pl.ds((t - 8) * 16, 16)]
                    u = jnp.where(IQ[q] == curv, gxt * (-EPS), -gxt)
                    r = jnp.minimum(rn[q], 255)
                    gv = plsc.load_gather(gmb, [r])
                    s = u + gv
                    better = (s > best) | ((s == best) & (r < bsr))
                    best = jnp.where(better, s, best)
                    bsr = jnp.where(better, r, bsr)
                    btk = jnp.where(better, IQ[q], btk)
            m = jnp.max(best)
            mr = jnp.min(jnp.where(best == m, bsr, big))
            tok = jnp.min(jnp.where((best == m) & (bsr == mr), btk, big))
            return jnp.where(lane == j, tok, toks), bstar

        carry = lax.fori_loop(0, 8, row_fn, (zeros16, 512))
        toks = carry[0]
        tokv[pl.ds(0, 16)] = toks
        tokv[pl.ds(16, 16)] = zeros16
        cps = []
        for t in range(8):
            tk = tokv[pl.ds(t, 16)][0]
            cps.append(pltpu.async_copy(em_hbm.at[pl.ds(tk, 1)],
                                        embr.at[pl.ds(t, 1)], sem))
        for c in cps:
            c.wait()
        pltpu.sync_copy(embr, out_hbm.at[pl.ds(wid * 8, 8)])

    return sck(logits2d, gxflat, hrow_arr, cur_arr, gmb_pad, embed_weight)


def kernel(gx, logits, embed_weight, output_ids, prompt_length):
    B, S, Vn = gx.shape
    E = embed_weight.shape[1]
    G = S - 8
    start = jnp.asarray(prompt_length, dtype=jnp.int32)

    rows = jnp.arange(B * G, dtype=jnp.int32)
    hrow_arr = (rows // G) * S + start + (rows % G)
    cur_arr = output_ids.reshape(B * S)[hrow_arr]

    g = jax.random.gumbel(jax.random.key(42), (B * G, K_VAL), jnp.float32)
    gmb_pad = jnp.concatenate(
        [g, jnp.full((B * G, 256 - K_VAL), -jnp.inf, jnp.float32)], axis=1)

    cur_embeds = embed_weight[:256] if _STAGE == 0 else _sc_sampler(
        logits.reshape(B * S, Vn), gx.reshape(B * S * Vn),
        hrow_arr, cur_arr, gmb_pad, embed_weight)

    bias = _bias_pallas(cur_embeds, embed_weight)
    return bias.reshape(B, G, Vn)


# speedup vs baseline: 1.1340x; 1.1340x over previous
import functools

import jax
import jax.numpy as jnp
from jax import lax
from jax.experimental import pallas as pl
from jax.experimental.pallas import tpu as pltpu
from jax.experimental.pallas import tpu_sc as plsc

EPS = 1e-10
K_VAL = 250
WEIGHT_VAL = 8.0

V = 100000
NV = V // 16
CAP = 3072
MININT = -2147483648

TV = 2048

_STAGE = 4


def _bias_body(e_ref, w_ref, o_ref):
    e = e_ref[...]
    w = w_ref[...]
    t1 = jnp.sum(w * w, axis=1)
    t3 = jnp.sum(e * e, axis=1)
    t2 = lax.dot_general(e, w, (((1,), (1,)), ((), ())),
                         preferred_element_type=jnp.float32)
    o_ref[...] = (2.0 * WEIGHT_VAL) * t2 \
        - WEIGHT_VAL * t1[None, :] - WEIGHT_VAL * t3[:, None]


def _bias_pallas(cur_embeds, embed_weight):
    R, E = cur_embeds.shape
    Vn = embed_weight.shape[0]
    return pl.pallas_call(
        _bias_body,
        grid=(pl.cdiv(Vn, TV),),
        in_specs=[
            pl.BlockSpec((R, E), lambda i: (0, 0)),
            pl.BlockSpec((TV, E), lambda i: (i, 0)),
        ],
        out_specs=pl.BlockSpec((R, TV), lambda i: (0, i)),
        out_shape=jax.ShapeDtypeStruct((R, Vn), jnp.float32),
    )(cur_embeds, embed_weight)


def _sc_sampler(logits2d, gxflat, hrow_arr, cur_arr, gmb_pad, embed_weight):
    mesh = plsc.VectorSubcoreMesh(core_axis_name="c", subcore_axis_name="s")

    @functools.partial(
        pl.kernel,
        mesh=mesh,
        compiler_params=pltpu.CompilerParams(needs_layout_passes=False),
        out_type=jax.ShapeDtypeStruct((256, 64), jnp.float32),
        scratch_types=[
            pltpu.VMEM((V,), jnp.float32),
            pltpu.VMEM((8192,), jnp.int32),
            pltpu.VMEM((CAP + 64,), jnp.int32),
            pltpu.VMEM((CAP + 64,), jnp.int32),
            pltpu.VMEM((272,), jnp.int32),
            pltpu.VMEM((272,), jnp.int32),
            pltpu.VMEM((128,), jnp.int32),
            pltpu.VMEM((128,), jnp.int32),
            pltpu.VMEM((128,), jnp.float32),
            pltpu.VMEM((128,), jnp.float32),
            pltpu.VMEM((256,), jnp.float32),
            pltpu.VMEM((272,), jnp.int32),
            pltpu.VMEM((272,), jnp.int32),
            pltpu.VMEM((32,), jnp.int32),
            pltpu.VMEM((8, 64), jnp.float32),
            pltpu.SMEM((8,), jnp.int32),
            pltpu.SemaphoreType.DMA,
        ],
    )
    def sck(lg_hbm, gx_hbm, hr_hbm, cu_hbm, gm_hbm, em_hbm, out_hbm,
            row_buf, hist, cks, cidx, Kb, Ib, ixa, ixb, gxa, gxb,
            gmb, hrv, crv, tokv, embr, cnt, sem):
        wid = lax.axis_index("s") * 2 + lax.axis_index("c")
        lane = lax.broadcasted_iota(jnp.int32, (16,), 0)
        zeros16 = jnp.zeros((16,), jnp.int32)
        ones16 = jnp.ones((16,), jnp.int32)
        pltpu.sync_copy(hr_hbm, hrv.at[pl.ds(0, 256)])
        pltpu.sync_copy(cu_hbm, crv.at[pl.ds(0, 256)])

        def keyize(v):
            b = lax.bitcast_convert_type(v, jnp.int32)
            m = lax.shift_right_logical(lax.shift_right_arithmetic(b, 31), 1)
            ks = b ^ m
            t9 = lax.shift_right_logical(ks, 23) ^ 256
            return ks, t9

        def row_fn(j, carry):
            toks, guess = carry
            row = wid * 8 + j
            hrow = hrv[pl.ds(row, 16)][0]
            pltpu.sync_copy(lg_hbm.at[hrow], row_buf)
            pltpu.sync_copy(gm_hbm.at[row], gmb)

            def zb(i, _):
                for u in range(8):
                    hist[pl.ds((i * 8 + u) * 16, 16)] = zeros16
                return 0
            lax.fori_loop(0, 64, zb, 0, unroll=False)

            bgv = jnp.full((16,), guess, jnp.int32)

            def fused_g(i, offv):
                kss, msks = [], []
                for u in range(8):
                    v = row_buf[pl.ds((i * 8 + u) * 16, 16)]
                    ks, t9 = keyize(v)
                    plsc.addupdate_scatter(hist, [t9 * 16 + lane], ones16)
                    kss.append(ks)
                    msks.append(t9 >= bgv)
                anym = ((msks[0] | msks[1]) | (msks[2] | msks[3])) | \
                       ((msks[4] | msks[5]) | (msks[6] | msks[7]))

                def dostore(_):
                    o = offv[0]
                    for u in range(8):
                        oc = jnp.minimum(o, CAP)
                        plsc.store_compressed(cks.at[pl.ds(oc, 16)], kss[u],
                                              mask=msks[u])
                        plsc.store_compressed(cidx.at[pl.ds(oc, 16)],
                                              (i * 8 + u) * 16 + lane,
                                              mask=msks[u])
                        o = o + jnp.sum(msks[u].astype(jnp.int32))
                    return 0
                lax.cond(jnp.any(anym), dostore, lambda _: 0, 0)
                upd = (plsc.all_reduce_population_count(msks[0])
                       + plsc.all_reduce_population_count(msks[1])) + \
                      (plsc.all_reduce_population_count(msks[2])
                       + plsc.all_reduce_population_count(msks[3])) + \
                      ((plsc.all_reduce_population_count(msks[4])
                        + plsc.all_reduce_population_count(msks[5])) +
                       (plsc.all_reduce_population_count(msks[6])
                        + plsc.all_reduce_population_count(msks[7])))
                return offv + upd
            offv = lax.fori_loop(0, NV // 8, fused_g, zeros16)
            for i4 in (NV // 8 * 8, NV // 8 * 8 + 1):
                v = row_buf[pl.ds(i4 * 16, 16)]
                ks, t9 = keyize(v)
                plsc.addupdate_scatter(hist, [t9 * 16 + lane], ones16)
                msk = t9 >= bgv
                oc = jnp.minimum(offv[0], CAP)
                plsc.store_compressed(cks.at[pl.ds(oc, 16)], ks, mask=msk)
                plsc.store_compressed(cidx.at[pl.ds(oc, 16)], i4 * 16 + lane,
                                      mask=msk)
                offv = offv + plsc.all_reduce_population_count(msk)

            def ga(t, c):
                cum, gst, gcb = c
                g = 31 - t
                acc = zeros16
                for bb in range(16):
                    acc = acc + hist[pl.ds((g * 16 + bb) * 16, 16)]
                tg = jnp.sum(acc)
                cumn = cum + tg
                hit = (cum < K_VAL) & (cumn >= K_VAL)
                gst = jnp.where(hit, g, gst)
                gcb = jnp.where(hit, cum, gcb)
                return cumn, gst, gcb
            _, gst, gcb = lax.fori_loop(0, 32, ga, (0, 0, 0))

            def gb(t, c):
                cum, bst, cab = c
                bb = gst * 16 + (15 - t)
                cc = jnp.sum(hist[pl.ds(bb * 16, 16)])
                cumn = cum + cc
                hit = (cum < K_VAL) & (cumn >= K_VAL)
                bst = jnp.where(hit, bb, bst)
                cab = jnp.where(hit, cum, cab)
                return cumn, bst, cab
            _, bstar, cnt_above = lax.fori_loop(0, 16, gb, (gcb, 0, 0))

            bsv = jnp.full((16,), bstar, jnp.int32)

            def cp_exact(i, offv2):
                v = row_buf[pl.ds(i * 16, 16)]
                ks, t9 = keyize(v)
                msk = t9 >= bsv
                offc = jnp.minimum(offv2[0], CAP)
                plsc.store_compressed(cks.at[pl.ds(offc, 16)], ks, mask=msk)
                plsc.store_compressed(cidx.at[pl.ds(offc, 16)], i * 16 + lane,
                                      mask=msk)
                return offv2 + plsc.all_reduce_population_count(msk)

            bad = (bstar < guess) | (offv[0] > CAP)
            cnt_f = lax.cond(
                bad,
                lambda _: lax.fori_loop(0, NV, cp_exact, zeros16, unroll=4)[0],
                lambda _: offv[0], 0)
            C = jnp.minimum(cnt_f, CAP)
            Cv = jnp.full((16,), C, jnp.int32)
            nvc = lax.shift_right_logical(C + 15, 4)
            nvc2 = lax.shift_right_logical(C + 31, 5)

            pfx = bstar
            pshift = 23
            cab = cnt_above
            for width in (5, 5, 5, 4, 4):
                shift = pshift - width
                nb = 1 << width
                for u in range(nb):
                    hist[pl.ds(u * 16, 16)] = zeros16
                pfxv = jnp.full((16,), pfx, jnp.int32)

                def rf(i, _, pfxv=pfxv, pshift=pshift, shift=shift, nb=nb):
                    for u in range(2):
                        ks = cks[pl.ds((i * 2 + u) * 16, 16)]
                        ku = ks ^ MININT
                        gi = ((i * 2 + u) * 16 + lane) < Cv
                        match = (lax.shift_right_logical(ku, pshift) == pfxv) \
                            & gi
                        bins = lax.shift_right_logical(ku, shift) & (nb - 1)
                        plsc.addupdate_scatter(hist, [bins * 16 + lane],
                                               ones16, mask=match)
                    return 0
                lax.fori_loop(0, nvc2, rf, 0)

                def sc2(t, c, nb=nb):
                    cum, bst, cab2 = c
                    bb = (nb - 1) - t
                    cc = jnp.sum(hist[pl.ds(bb * 16, 16)])
                    cumn = cum + cc
                    hit = (cum < K_VAL) & (cumn >= K_VAL)
                    bst = jnp.where(hit, bb, bst)
                    cab2 = jnp.where(hit, cum, cab2)
                    return cumn, bst, cab2
                _, bst, cab = lax.fori_loop(0, nb, sc2, (cab, 0, 0))
                pfx = lax.shift_left(pfx, width) | bst
                pshift = shift
            kth_ks = pfx ^ MININT
            need = K_VAL - cab

            for t in range(17):
                Kb[pl.ds(t * 16, 16)] = jnp.full((16,), MININT, jnp.int32)
                Ib[pl.ds(t * 16, 16)] = zeros16
            cnt[1] = 0
            cnt[2] = 0
            kthv = jnp.full((16,), kth_ks, jnp.int32)

            def ex(i, _):
                for u in range(2):
                    ks = cks[pl.ds((i * 2 + u) * 16, 16)]
                    iv = cidx[pl.ds((i * 2 + u) * 16, 16)]
                    gi = ((i * 2 + u) * 16 + lane) < Cv
                    gt = (ks > kthv) & gi
                    eq = (ks == kthv) & gi
                    eqi = eq.astype(jnp.int32)
                    pre = plsc.cumsum(eqi) - eqi
                    take = eq & ((pre + cnt[2]) < need)
                    fm = gt | take
                    off = cnt[1]
                    plsc.store_compressed(Kb.at[pl.ds(off, 16)], ks, mask=fm)
                    plsc.store_compressed(Ib.at[pl.ds(off, 16)], iv, mask=fm)
                    cnt[1] = off + jnp.sum(fm.astype(jnp.int32))
                    cnt[2] = cnt[2] + jnp.sum(eqi)
                return 0
            lax.fori_loop(0, nvc2, ex, 0, unroll=False)

            gxbase = jnp.full((16,), hrow * V, jnp.int32)
            for t in range(8):
                ixa[pl.ds(t * 16, 16)] = Ib[pl.ds(t * 16, 16)] + gxbase
            for t in range(8, 16):
                ixb[pl.ds((t - 8) * 16, 16)] = Ib[pl.ds(t * 16, 16)] + gxbase
            cpa = pltpu.async_copy(gx_hbm.at[ixa], gxa, sem)
            cpb = pltpu.async_copy(gx_hbm.at[ixb], gxb, sem)

            curv = jnp.full((16,), crv[pl.ds(row, 16)][0], jnp.int32)
            best = jnp.full((16,), -jnp.inf, jnp.float32)
            bsr = jnp.full((16,), 1 << 30, jnp.int32)
            btk = zeros16
            big = jnp.int32(1 << 30)
            for qb in range(4):
                KQ = [Kb[pl.ds((qb * 4 + q) * 16, 16)] for q in range(4)]
                IQ = [Ib[pl.ds((qb * 4 + q) * 16, 16)] for q in range(4)]

                def apb(t, rn, KQ=KQ, IQ=IQ):
                    kv2 = Kb[pl.ds(t * 2, 16)]
                    iv2 = Ib[pl.ds(t * 2, 16)]
                    kpv = jnp.full((16,), kv2[0], jnp.int32)
                    ipv = jnp.full((16,), iv2[0], jnp.int32)
                    kpv2 = jnp.full((16,), kv2[1], jnp.int32)
                    ipv2 = jnp.full((16,), iv2[1], jnp.int32)
                    out = []
                    for q in range(4):
                        gt = (kpv > KQ[q]).astype(jnp.int32)
                        eq = ((kpv == KQ[q]) & (ipv < IQ[q])).astype(jnp.int32)
                        gt2 = (kpv2 > KQ[q]).astype(jnp.int32)
                        eq2 = ((kpv2 == KQ[q]) &
                               (ipv2 < IQ[q])).astype(jnp.int32)
                        out.append(rn[q] + (gt + eq) + (gt2 + eq2))
                    return tuple(out)
                rn = lax.fori_loop(0, 128, apb, (zeros16,) * 4)
                if qb == 0:
                    cpa.wait()
                    cpb.wait()
                for q in range(4):
                    t = qb * 4 + q
                    if t < 8:
                        gxt = gxa[pl.ds(t * 16, 16)]
                    else:
                        gxt = gxb[pl.ds((t - 8) * 16, 16)]
                    u = jnp.where(IQ[q] == curv, gxt * (-EPS), -gxt)
                    r = jnp.minimum(rn[q], 255)
                    gv = plsc.load_gather(gmb, [r])
                    s = u + gv
                    better = (s > best) | ((s == best) & (r < bsr))
                    best = jnp.where(better, s, best)
                    bsr = jnp.where(better, r, bsr)
                    btk = jnp.where(better, IQ[q], btk)
            m = jnp.max(best)
            mr = jnp.min(jnp.where(best == m, bsr, big))
            tok = jnp.min(jnp.where((best == m) & (bsr == mr), btk, big))
            return jnp.where(lane == j, tok, toks), bstar

        carry = lax.fori_loop(0, 8, row_fn, (zeros16, 512))
        toks = carry[0]
        tokv[pl.ds(0, 16)] = toks
        tokv[pl.ds(16, 16)] = zeros16
        cps = []
        for t in range(8):
            tk = tokv[pl.ds(t, 16)][0]
            cps.append(pltpu.async_copy(em_hbm.at[pl.ds(tk, 1)],
                                        embr.at[pl.ds(t, 1)], sem))
        for c in cps:
            c.wait()
        pltpu.sync_copy(embr, out_hbm.at[pl.ds(wid * 8, 8)])

    return sck(logits2d, gxflat, hrow_arr, cur_arr, gmb_pad, embed_weight)


def kernel(gx, logits, embed_weight, output_ids, prompt_length):
    B, S, Vn = gx.shape
    E = embed_weight.shape[1]
    G = S - 8
    start = jnp.asarray(prompt_length, dtype=jnp.int32)

    rows = jnp.arange(B * G, dtype=jnp.int32)
    hrow_arr = (rows // G) * S + start + (rows % G)
    cur_arr = output_ids.reshape(B * S)[hrow_arr]

    g = jax.random.gumbel(jax.random.key(42), (B * G, K_VAL), jnp.float32)
    gmb_pad = jnp.concatenate(
        [g, jnp.full((B * G, 256 - K_VAL), -jnp.inf, jnp.float32)], axis=1)

    cur_embeds = embed_weight[:256] if _STAGE == 0 else _sc_sampler(
        logits.reshape(B * S, Vn), gx.reshape(B * S * Vn),
        hrow_arr, cur_arr, gmb_pad, embed_weight)

    bias = _bias_pallas(cur_embeds, embed_weight)
    return bias.reshape(B, G, Vn)


# speedup vs baseline: 1.7334x; 1.5285x over previous
import functools

import jax
import jax.numpy as jnp
from jax import lax
from jax.experimental import pallas as pl
from jax.experimental.pallas import tpu as pltpu
from jax.experimental.pallas import tpu_sc as plsc

EPS = 1e-10
K_VAL = 250
WEIGHT_VAL = 8.0

V = 100000
NV = V // 16
CAP = 3072
MININT = -2147483648

TV = 2048

_STAGE = 4


def _bias_body(e_ref, w_ref, o_ref):
    e = e_ref[...]
    w = w_ref[...]
    t1 = jnp.sum(w * w, axis=1)
    t3 = jnp.sum(e * e, axis=1)
    t2 = lax.dot_general(e, w, (((1,), (1,)), ((), ())),
                         preferred_element_type=jnp.float32)
    o_ref[...] = (2.0 * WEIGHT_VAL) * t2 \
        - WEIGHT_VAL * t1[None, :] - WEIGHT_VAL * t3[:, None]


def _bias_pallas(cur_embeds, embed_weight):
    R, E = cur_embeds.shape
    Vn = embed_weight.shape[0]
    return pl.pallas_call(
        _bias_body,
        grid=(pl.cdiv(Vn, TV),),
        in_specs=[
            pl.BlockSpec((R, E), lambda i: (0, 0)),
            pl.BlockSpec((TV, E), lambda i: (i, 0)),
        ],
        out_specs=pl.BlockSpec((R, TV), lambda i: (0, i)),
        out_shape=jax.ShapeDtypeStruct((R, Vn), jnp.float32),
    )(cur_embeds, embed_weight)


def _sc_sampler(logits2d, gxflat, hrow_arr, cur_arr, gmb_pad, embed_weight):
    mesh = plsc.VectorSubcoreMesh(core_axis_name="c", subcore_axis_name="s")

    @functools.partial(
        pl.kernel,
        mesh=mesh,
        compiler_params=pltpu.CompilerParams(needs_layout_passes=False),
        out_type=jax.ShapeDtypeStruct((256, 64), jnp.float32),
        scratch_types=[
            pltpu.VMEM((V,), jnp.float32),
            pltpu.VMEM((8192,), jnp.int32),
            pltpu.VMEM((CAP + 64,), jnp.int32),
            pltpu.VMEM((CAP + 64,), jnp.int32),
            pltpu.VMEM((272,), jnp.int32),
            pltpu.VMEM((272,), jnp.int32),
            pltpu.VMEM((128,), jnp.int32),
            pltpu.VMEM((128,), jnp.int32),
            pltpu.VMEM((128,), jnp.float32),
            pltpu.VMEM((128,), jnp.float32),
            pltpu.VMEM((256,), jnp.float32),
            pltpu.VMEM((272,), jnp.int32),
            pltpu.VMEM((272,), jnp.int32),
            pltpu.VMEM((32,), jnp.int32),
            pltpu.VMEM((8, 64), jnp.float32),
            pltpu.SMEM((8,), jnp.int32),
            pltpu.SemaphoreType.DMA,
        ],
    )
    def sck(lg_hbm, gx_hbm, hr_hbm, cu_hbm, gm_hbm, em_hbm, out_hbm,
            row_buf, hist, cks, cidx, Kb, Ib, ixa, ixb, gxa, gxb,
            gmb, hrv, crv, tokv, embr, cnt, sem):
        wid = lax.axis_index("s") * 2 + lax.axis_index("c")
        lane = lax.broadcasted_iota(jnp.int32, (16,), 0)
        zeros16 = jnp.zeros((16,), jnp.int32)
        ones16 = jnp.ones((16,), jnp.int32)
        pltpu.sync_copy(hr_hbm, hrv.at[pl.ds(0, 256)])
        pltpu.sync_copy(cu_hbm, crv.at[pl.ds(0, 256)])

        def keyize(v):
            b = lax.bitcast_convert_type(v, jnp.int32)
            m = lax.shift_right_logical(lax.shift_right_arithmetic(b, 31), 1)
            ks = b ^ m
            t9 = lax.shift_right_logical(ks, 23) ^ 256
            return ks, t9

        def row_fn(j, carry):
            toks, guess = carry
            row = wid * 8 + j
            hrow = hrv[pl.ds(row, 16)][0]
            pltpu.sync_copy(lg_hbm.at[hrow], row_buf)
            pltpu.sync_copy(gm_hbm.at[row], gmb)

            bgv = jnp.full((16,), guess, jnp.int32)

            def app_g(i, offv, bv=bgv):
                kss, msks = [], []
                for u in range(8):
                    v = row_buf[pl.ds((i * 8 + u) * 16, 16)]
                    ks, t9 = keyize(v)
                    kss.append(ks)
                    msks.append(t9 >= bv)
                anym = ((msks[0] | msks[1]) | (msks[2] | msks[3])) | \
                       ((msks[4] | msks[5]) | (msks[6] | msks[7]))

                def dostore(_):
                    o = offv[0]
                    for u in range(8):
                        oc = jnp.minimum(o, CAP)
                        plsc.store_compressed(cks.at[pl.ds(oc, 16)], kss[u],
                                              mask=msks[u])
                        plsc.store_compressed(cidx.at[pl.ds(oc, 16)],
                                              (i * 8 + u) * 16 + lane,
                                              mask=msks[u])
                        o = o + jnp.sum(msks[u].astype(jnp.int32))
                    return 0
                lax.cond(jnp.any(anym), dostore, lambda _: 0, 0)
                upd = (plsc.all_reduce_population_count(msks[0])
                       + plsc.all_reduce_population_count(msks[1])) + \
                      (plsc.all_reduce_population_count(msks[2])
                       + plsc.all_reduce_population_count(msks[3])) + \
                      ((plsc.all_reduce_population_count(msks[4])
                        + plsc.all_reduce_population_count(msks[5])) +
                       (plsc.all_reduce_population_count(msks[6])
                        + plsc.all_reduce_population_count(msks[7])))
                return offv + upd

            def app_tail(offv, bv):
                for i4 in (NV // 8 * 8, NV // 8 * 8 + 1):
                    v = row_buf[pl.ds(i4 * 16, 16)]
                    ks, t9 = keyize(v)
                    msk = t9 >= bv
                    oc = jnp.minimum(offv[0], CAP)
                    plsc.store_compressed(cks.at[pl.ds(oc, 16)], ks, mask=msk)
                    plsc.store_compressed(cidx.at[pl.ds(oc, 16)],
                                          i4 * 16 + lane, mask=msk)
                    offv = offv + plsc.all_reduce_population_count(msk)
                return offv

            offv = app_tail(lax.fori_loop(0, NV // 8, app_g, zeros16), bgv)
            cnt0 = offv[0]

            def zb(i, _):
                for u in range(8):
                    hist[pl.ds((i * 8 + u) * 16, 16)] = zeros16
                return 0
            lax.fori_loop(0, 64, zb, 0, unroll=False)

            def scan512(cum0):
                def ga(t, c):
                    cum, gst, gcb = c
                    g = 31 - t
                    acc = zeros16
                    for bb in range(16):
                        acc = acc + hist[pl.ds((g * 16 + bb) * 16, 16)]
                    tg = jnp.sum(acc)
                    cumn = cum + tg
                    hit = (cum < K_VAL) & (cumn >= K_VAL)
                    gst = jnp.where(hit, g, gst)
                    gcb = jnp.where(hit, cum, gcb)
                    return cumn, gst, gcb
                _, gst, gcb = lax.fori_loop(0, 32, ga, (cum0, 0, 0))

                def gb(t, c):
                    cum, bst, cab = c
                    bb = gst * 16 + (15 - t)
                    cc = jnp.sum(hist[pl.ds(bb * 16, 16)])
                    cumn = cum + cc
                    hit = (cum < K_VAL) & (cumn >= K_VAL)
                    bst = jnp.where(hit, bb, bst)
                    cab = jnp.where(hit, cum, cab)
                    return cumn, bst, cab
                _, bst, cab = lax.fori_loop(0, 16, gb, (gcb, 0, 0))
                return bst, cab

            bad = (cnt0 < K_VAL) | (cnt0 > CAP)

            def fallback(_):
                def fh(i, _):
                    for u in range(2):
                        v = row_buf[pl.ds((i * 2 + u) * 16, 16)]
                        _ks, t9 = keyize(v)
                        plsc.addupdate_scatter(hist, [t9 * 16 + lane], ones16)
                    return 0
                lax.fori_loop(0, NV // 2, fh, 0)
                bst, _cab = scan512(0)
                bsv = jnp.full((16,), bst, jnp.int32)

                def cp_exact(i, offv2):
                    v = row_buf[pl.ds(i * 16, 16)]
                    ks, t9 = keyize(v)
                    msk = t9 >= bsv
                    offc = jnp.minimum(offv2[0], CAP)
                    plsc.store_compressed(cks.at[pl.ds(offc, 16)], ks,
                                          mask=msk)
                    plsc.store_compressed(cidx.at[pl.ds(offc, 16)],
                                          i * 16 + lane, mask=msk)
                    return offv2 + plsc.all_reduce_population_count(msk)
                cnt2 = lax.fori_loop(0, NV, cp_exact, zeros16, unroll=4)[0]
                lax.fori_loop(0, 64, zb, 0, unroll=False)
                return cnt2, bst

            cnt_f, gbucket = lax.cond(bad, fallback,
                                      lambda _: (cnt0, guess), 0)
            C = jnp.minimum(cnt_f, CAP)
            Cv = jnp.full((16,), C, jnp.int32)
            nvc = lax.shift_right_logical(C + 15, 4)
            nvc2 = lax.shift_right_logical(C + 31, 5)

            def h0(i, _):
                for u in range(2):
                    ks = cks[pl.ds((i * 2 + u) * 16, 16)]
                    t9 = lax.shift_right_logical(ks, 23) ^ 256
                    gi = ((i * 2 + u) * 16 + lane) < Cv
                    plsc.addupdate_scatter(hist, [t9 * 16 + lane], ones16,
                                           mask=gi)
                return 0
            lax.fori_loop(0, nvc2, h0, 0)
            bstar, cnt_above = scan512(0)

            pfx = bstar
            pshift = 23
            cab = cnt_above
            for width in (5, 5, 5, 4, 4):
                shift = pshift - width
                nb = 1 << width
                for u in range(nb):
                    hist[pl.ds(u * 16, 16)] = zeros16
                pfxv = jnp.full((16,), pfx, jnp.int32)

                def rf(i, _, pfxv=pfxv, pshift=pshift, shift=shift, nb=nb):
                    for u in range(2):
                        ks = cks[pl.ds((i * 2 + u) * 16, 16)]
                        ku = ks ^ MININT
                        gi = ((i * 2 + u) * 16 + lane) < Cv
                        match = (lax.shift_right_logical(ku, pshift) == pfxv) \
                            & gi
                        bins = lax.shift_right_logical(ku, shift) & (nb - 1)
                        plsc.addupdate_scatter(hist, [bins * 16 + lane],
                                               ones16, mask=match)
                    return 0
                lax.fori_loop(0, nvc2, rf, 0)

                def sc2(t, c, nb=nb):
                    cum, bst, cab2 = c
                    bb = (nb - 1) - t
                    cc = jnp.sum(hist[pl.ds(bb * 16, 16)])
                    cumn = cum + cc
                    hit = (cum < K_VAL) & (cumn >= K_VAL)
                    bst = jnp.where(hit, bb, bst)
                    cab2 = jnp.where(hit, cum, cab2)
                    return cumn, bst, cab2
                _, bst, cab = lax.fori_loop(0, nb, sc2, (cab, 0, 0))
                pfx = lax.shift_left(pfx, width) | bst
                pshift = shift
            kth_ks = pfx ^ MININT
            need = K_VAL - cab

            for t in range(17):
                Kb[pl.ds(t * 16, 16)] = jnp.full((16,), MININT, jnp.int32)
                Ib[pl.ds(t * 16, 16)] = zeros16
            cnt[1] = 0
            cnt[2] = 0
            kthv = jnp.full((16,), kth_ks, jnp.int32)

            def ex(i, _):
                for u in range(2):
                    ks = cks[pl.ds((i * 2 + u) * 16, 16)]
                    iv = cidx[pl.ds((i * 2 + u) * 16, 16)]
                    gi = ((i * 2 + u) * 16 + lane) < Cv
                    gt = (ks > kthv) & gi
                    eq = (ks == kthv) & gi
                    eqi = eq.astype(jnp.int32)
                    pre = plsc.cumsum(eqi) - eqi
                    take = eq & ((pre + cnt[2]) < need)
                    fm = gt | take
                    off = cnt[1]
                    plsc.store_compressed(Kb.at[pl.ds(off, 16)], ks, mask=fm)
                    plsc.store_compressed(Ib.at[pl.ds(off, 16)], iv, mask=fm)
                    cnt[1] = off + jnp.sum(fm.astype(jnp.int32))
                    cnt[2] = cnt[2] + jnp.sum(eqi)
                return 0
            lax.fori_loop(0, nvc2, ex, 0, unroll=False)

            gxbase = jnp.full((16,), hrow * V, jnp.int32)
            for t in range(8):
                ixa[pl.ds(t * 16, 16)] = Ib[pl.ds(t * 16, 16)] + gxbase
            for t in range(8, 16):
                ixb[pl.ds((t - 8) * 16, 16)] = Ib[pl.ds(t * 16, 16)] + gxbase
            cpa = pltpu.async_copy(gx_hbm.at[ixa], gxa, sem)
            cpb = pltpu.async_copy(gx_hbm.at[ixb], gxb, sem)

            curv = jnp.full((16,), crv[pl.ds(row, 16)][0], jnp.int32)
            best = jnp.full((16,), -jnp.inf, jnp.float32)
            bsr = jnp.full((16,), 1 << 30, jnp.int32)
            btk = zeros16
            big = jnp.int32(1 << 30)
            for qb in range(4):
                KQ = [Kb[pl.ds((qb * 4 + q) * 16, 16)] for q in range(4)]
                IQ = [Ib[pl.ds((qb * 4 + q) * 16, 16)] for q in range(4)]

                def apb(t, rn, KQ=KQ, IQ=IQ):
                    kv2 = Kb[pl.ds(t * 2, 16)]
                    iv2 = Ib[pl.ds(t * 2, 16)]
                    kpv = jnp.full((16,), kv2[0], jnp.int32)
                    ipv = jnp.full((16,), iv2[0], jnp.int32)
                    kpv2 = jnp.full((16,), kv2[1], jnp.int32)
                    ipv2 = jnp.full((16,), iv2[1], jnp.int32)
                    out = []
                    for q in range(4):
                        gt = (kpv > KQ[q]).astype(jnp.int32)
                        eq = ((kpv == KQ[q]) & (ipv < IQ[q])).astype(jnp.int32)
                        gt2 = (kpv2 > KQ[q]).astype(jnp.int32)
                        eq2 = ((kpv2 == KQ[q]) &
                               (ipv2 < IQ[q])).astype(jnp.int32)
                        out.append(rn[q] + (gt + eq) + (gt2 + eq2))
                    return tuple(out)
                rn = lax.fori_loop(0, 128, apb, (zeros16,) * 4)
                if qb == 0:
                    cpa.wait()
                    cpb.wait()
                for q in range(4):
                    t = qb * 4 + q
                    if t < 8:
                        gxt = gxa[pl.ds(t * 16, 16)]
                    else:
                        gxt = gxb[pl.ds((t - 8) * 16, 16)]
                    u = jnp.where(IQ[q] == curv, gxt * (-EPS), -gxt)
                    r = jnp.minimum(rn[q], 255)
                    gv = plsc.load_gather(gmb, [r])
                    s = u + gv
                    better = (s > best) | ((s == best) & (r < bsr))
                    best = jnp.where(better, s, best)
                    bsr = jnp.where(better, r, bsr)
                    btk = jnp.where(better, IQ[q], btk)
            m = jnp.max(best)
            mr = jnp.min(jnp.where(best == m, bsr, big))
            tok = jnp.min(jnp.where((best == m) & (bsr == mr), btk, big))
            return jnp.where(lane == j, tok, toks), bstar

        carry = lax.fori_loop(0, 8, row_fn, (zeros16, jnp.int32(384)))
        toks = carry[0]
        tokv[pl.ds(0, 16)] = toks
        tokv[pl.ds(16, 16)] = zeros16
        cps = []
        for t in range(8):
            tk = tokv[pl.ds(t, 16)][0]
            cps.append(pltpu.async_copy(em_hbm.at[pl.ds(tk, 1)],
                                        embr.at[pl.ds(t, 1)], sem))
        for c in cps:
            c.wait()
        pltpu.sync_copy(embr, out_hbm.at[pl.ds(wid * 8, 8)])

    return sck(logits2d, gxflat, hrow_arr, cur_arr, gmb_pad, embed_weight)


def kernel(gx, logits, embed_weight, output_ids, prompt_length):
    B, S, Vn = gx.shape
    E = embed_weight.shape[1]
    G = S - 8
    start = jnp.asarray(prompt_length, dtype=jnp.int32)

    rows = jnp.arange(B * G, dtype=jnp.int32)
    hrow_arr = (rows // G) * S + start + (rows % G)
    cur_arr = output_ids.reshape(B * S)[hrow_arr]

    g = jax.random.gumbel(jax.random.key(42), (B * G, K_VAL), jnp.float32)
    gmb_pad = jnp.concatenate(
        [g, jnp.full((B * G, 256 - K_VAL), -jnp.inf, jnp.float32)], axis=1)

    cur_embeds = embed_weight[:256] if _STAGE == 0 else _sc_sampler(
        logits.reshape(B * S, Vn), gx.reshape(B * S * Vn),
        hrow_arr, cur_arr, gmb_pad, embed_weight)

    bias = _bias_pallas(cur_embeds, embed_weight)
    return bias.reshape(B, G, Vn)
